# Initial kernel scaffold; baseline (speedup 1.0000x reference)
import jax
import jax.numpy as jnp
from jax.experimental import pallas as pl


def kernel(z, W):
    raise NotImplementedError("write your pallas kernel here")


# speedup vs baseline: 1.3145x; 1.3145x over previous
import functools

import jax
import jax.numpy as jnp
from jax import lax
from jax.experimental import pallas as pl
from jax.experimental.pallas import tpu as pltpu
from jax.experimental.pallas import tpu_sc as plsc

NUM_EMBEDDINGS = 8192
EMBEDDING_DIM = 32

ROW_BLOCK = 256


def _rowsum32_lanes(x2):
    a = x2[:, 0:8] + x2[:, 8:16] + x2[:, 16:24] + x2[:, 24:32]
    a = a[:, 0:4] + a[:, 4:8]
    a = a[:, 0:2] + a[:, 2:4]
    return a[:, 0:1] + a[:, 1:2]


def _colsum32_sublanes(x2):
    a = x2[0:8, :] + x2[8:16, :] + x2[16:24, :] + x2[24:32, :]
    a = a[0:4, :] + a[4:8, :]
    a = a[0:2, :] + a[2:4, :]
    return a[0:1, :] + a[1:2, :]


def _argmin_body(z_ref, w_ref, wt_ref, idx_ref):
    z = z_ref[...]
    w = w_ref[...]
    wt = wt_ref[...]
    dots = lax.dot_general(
        z, w, (((1,), (1,)), ((), ())),
        preferred_element_type=jnp.float32)
    z2 = _rowsum32_lanes(z * z)
    w2 = _colsum32_sublanes(wt * wt)
    dist = (z2 + w2) - 2.0 * dots
    mv = jnp.min(dist, axis=1, keepdims=True)
    iota = lax.broadcasted_iota(jnp.int32, dist.shape, 1)
    idx = jnp.min(jnp.where(dist == mv, iota, jnp.int32(NUM_EMBEDDINGS)),
                  axis=1)
    idx_ref[...] = idx[None, None, :]


def _compute_indices(z_flat, W, Wt):
    n = z_flat.shape[0]
    nb = n // ROW_BLOCK
    out = pl.pallas_call(
        _argmin_body,
        grid=(nb,),
        in_specs=[
            pl.BlockSpec((ROW_BLOCK, EMBEDDING_DIM), lambda i: (i, 0)),
            pl.BlockSpec((NUM_EMBEDDINGS, EMBEDDING_DIM), lambda i: (0, 0)),
            pl.BlockSpec((EMBEDDING_DIM, NUM_EMBEDDINGS), lambda i: (0, 0)),
        ],
        out_specs=pl.BlockSpec((1, 1, ROW_BLOCK), lambda i: (i, 0, 0)),
        out_shape=jax.ShapeDtypeStruct((nb, 1, ROW_BLOCK), jnp.int32),
    )(z_flat, W, Wt)
    return out.reshape(n)


GATHER_CHUNK = 512


def _make_gather(n_rows):
    info = plsc.get_sparse_core_info()
    nw = info.num_cores * info.num_subcores
    b_per_w = n_rows // nw
    n_chunks = b_per_w // GATHER_CHUNK
    mesh = plsc.VectorSubcoreMesh(core_axis_name="c", subcore_axis_name="s")

    @functools.partial(
        pl.kernel, mesh=mesh,
        out_type=jax.ShapeDtypeStruct((n_rows, 128), jnp.float32),
        scratch_types=[
            pltpu.VMEM((b_per_w,), jnp.int32),
            pltpu.VMEM((GATHER_CHUNK, 128), jnp.float32),
            pltpu.SemaphoreType.DMA,
        ],
    )
    def gather(table_hbm, idx_hbm, out_hbm, idx_v, rows_v, sem):
        wid = lax.axis_index("s") * info.num_cores + lax.axis_index("c")
        base = wid * b_per_w
        pltpu.sync_copy(idx_hbm.at[pl.ds(base, b_per_w)], idx_v)
        for c in range(n_chunks):
            pltpu.async_copy(
                table_hbm.at[idx_v.at[pl.ds(c * GATHER_CHUNK, GATHER_CHUNK)]],
                rows_v, sem).wait()
            pltpu.sync_copy(
                rows_v,
                out_hbm.at[pl.ds(base + c * GATHER_CHUNK, GATHER_CHUNK)])

    return gather


def kernel(z, W):
    z_flat = z.reshape(-1, EMBEDDING_DIM)
    Wt = W.T
    idx = _compute_indices(z_flat, W, Wt)
    W_pad = jnp.pad(W, ((0, 0), (0, 128 - EMBEDDING_DIM)))
    quantized = _make_gather(z_flat.shape[0])(W_pad, idx)[:, :EMBEDDING_DIM]
    return quantized.reshape(z.shape), idx[:, None]
